# baseline (speedup 1.0000x reference)
import functools
import jax
import jax.numpy as jnp
from jax import lax
from jax.experimental import pallas as pl
from jax.experimental.pallas import tpu as pltpu
from jax.experimental.pallas import tpu_sc as plsc

B = 32
A = 20000
G = 16
L = 16
NV = A // L
CH = 400
NCH = A // CH
CHV = CH // L
IOU_THRESHOLD = 0.5
NEG_POS_RATIO = 3.0
ALPHA = 0.25
LN2 = 0.6931471805599453
NBINS = 64
NLEVELS = 3


def _plog(x):
    bits = lax.bitcast_convert_type(x, jnp.int32)
    e = lax.convert_element_type((bits >> 23) - 127, jnp.float32)
    m = lax.bitcast_convert_type(
        (bits & jnp.int32(0x7FFFFF)) | jnp.int32(0x3F800000), jnp.float32)
    f = m - 1.0
    z = f / (2.0 + f)
    z2 = z * z
    at = z * (1.0 + z2 * (1.0 / 3 + z2 * (1.0 / 5 + z2 * (1.0 / 7 + z2 / 9))))
    return (e + (2.0 / LN2) * at) * LN2


def _sc_body(bbox_hbm, conf_hbm, anch_hbm, gt_hbm, out_hbm,
             slab_v, biou_v, bidx_v, gim_v, gsm_v, gidx_v,
             gx1_v, gy1_v, gx2_v, gy2_v, conf_v, out_v, hist_v):
    wid = lax.axis_index("s") * 2 + lax.axis_index("c")
    iot = lax.iota(jnp.int32, L)

    pltpu.sync_copy(anch_hbm, slab_v)
    gbase = wid * (4 * G)
    pltpu.sync_copy(gt_hbm.at[pl.ds(gbase, G)], gx1_v)
    pltpu.sync_copy(gt_hbm.at[pl.ds(gbase + G, G)], gy1_v)
    pltpu.sync_copy(gt_hbm.at[pl.ds(gbase + 2 * G, G)], gx2_v)
    pltpu.sync_copy(gt_hbm.at[pl.ds(gbase + 3 * G, G)], gy2_v)

    gx1r, gy1r, gx2r, gy2r = gx1_v[:], gy1_v[:], gx2_v[:], gy2_v[:]
    gx1s = [gx1r[g] for g in range(G)]
    gy1s = [gy1r[g] for g in range(G)]
    gx2s = [gx2r[g] for g in range(G)]
    gy2s = [gy2r[g] for g in range(G)]
    ag7 = [(gx2s[g] - gx1s[g]) * (gy2s[g] - gy1s[g]) + 1e-7 for g in range(G)]

    for g in range(G):
        gim_v[g, :] = jnp.full((L,), -1.0, jnp.float32)
        gsm_v[g, :] = jnp.full((L,), 1.0, jnp.float32)
        gidx_v[g, :] = jnp.zeros((L,), jnp.int32)

    def p1(c, carry):
        sl = pl.ds(c * L, L)
        ax1 = slab_v[0, sl]
        ay1 = slab_v[1, sl]
        ax2 = slab_v[2, sl]
        ay2 = slab_v[3, sl]
        area_a = (ax2 - ax1) * (ay2 - ay1)
        aid = c * L + iot
        ib = jnp.full((L,), -1.0, jnp.float32)
        sb = jnp.full((L,), 1.0, jnp.float32)
        bidx = jnp.zeros((L,), jnp.int32)
        for g in range(G):
            iw = jnp.maximum(
                jnp.minimum(ax2, gx2s[g]) - jnp.maximum(ax1, gx1s[g]), 0.0)
            ih = jnp.maximum(
                jnp.minimum(ay2, gy2s[g]) - jnp.maximum(ay1, gy1s[g]), 0.0)
            inter = iw * ih
            s = area_a + ag7[g]
            upd = inter * sb > ib * s
            ib = jnp.where(upd, inter, ib)
            sb = jnp.where(upd, s, sb)
            bidx = jnp.where(upd, jnp.full((L,), g, jnp.int32), bidx)
            gim = gim_v[g, :]
            gsm = gsm_v[g, :]
            gu = inter * gsm > gim * s
            gim_v[g, :] = jnp.where(gu, inter, gim)
            gsm_v[g, :] = jnp.where(gu, s, gsm)
            gidx_v[g, :] = jnp.where(gu, aid, gidx_v[g, :])
        biou_v[sl] = jnp.where(3.0 * ib > sb, 1.0, 0.0)
        bidx_v[sl] = bidx
        return carry

    lax.fori_loop(0, NV, p1, jnp.int32(0))

    fids = jnp.zeros((L,), jnp.int32)
    for g in range(G):
        gim = gim_v[g, :]
        row = gim / (gsm_v[g, :] - gim)
        gi = gidx_v[g, :]
        gm = jnp.max(row)
        cand = jnp.where(row == gm, gi, jnp.full((L,), A, jnp.int32))
        fid = jnp.min(cand)
        fids = jnp.where(iot == g, fid, fids)
    plsc.store_scatter(biou_v, [fids], jnp.full((L,), 2.0, jnp.float32))

    pltpu.sync_copy(bbox_hbm.at[wid], slab_v)

    def p2(k, carry):
        acc_loc, acc_posl, acc_np = carry
        pltpu.sync_copy(conf_hbm.at[pl.ds(wid * A + k * CH, CH)], conf_v)

        def p2i(j, icarry):
            acc_loc, acc_posl, acc_np = icarry
            o = k * CH + j * L
            sl = pl.ds(o, L)
            slc = pl.ds(j * L, L)
            bi = biou_v[sl]
            bx = bidx_v[sl]
            pos = bi > IOU_THRESHOLD
            posf = jnp.where(pos, 1.0, 0.0)
            tx1 = plsc.load_gather(gx1_v, [bx])
            ty1 = plsc.load_gather(gy1_v, [bx])
            tx2 = plsc.load_gather(gx2_v, [bx])
            ty2 = plsc.load_gather(gy2_v, [bx])
            px1 = slab_v[0, sl]
            py1 = slab_v[1, sl]
            px2 = slab_v[2, sl]
            py2 = slab_v[3, sl]
            area_p = (px2 - px1) * (py2 - py1)
            area_t = (tx2 - tx1) * (ty2 - ty1)
            iw = jnp.maximum(jnp.minimum(px2, tx2) - jnp.maximum(px1, tx1), 0.0)
            ih = jnp.maximum(jnp.minimum(py2, ty2) - jnp.maximum(py1, ty1), 0.0)
            inter = iw * ih
            union = area_p + area_t - inter + 1e-7
            iou = inter / union
            dx = (px1 + px2) - (tx1 + tx2)
            dy = (py1 + py2) - (ty1 + ty2)
            rho2 = 0.25 * (dx * dx + dy * dy)
            ex = jnp.maximum(px2, tx2) - jnp.minimum(px1, tx1)
            ey = jnp.maximum(py2, ty2) - jnp.minimum(py1, ty1)
            c2 = ex * ex + ey * ey + 1e-7
            loc = jnp.minimum(1.0 - iou + rho2 / c2, 100.0)
            acc_loc = acc_loc + loc * posf
            lg = conf_v[slc]
            ez = jnp.exp(-jnp.abs(lg))
            ce = jnp.maximum(lg, 0.0) - lg * posf + _plog(1.0 + ez)
            p = jnp.where(lg >= 0.0, 1.0, ez) / (1.0 + ez)
            p_t = p * posf + (1.0 - p) * (1.0 - posf)
            alpha_t = ALPHA * posf + (1.0 - ALPHA) * (1.0 - posf)
            om = 1.0 - p_t
            acl = jnp.minimum(alpha_t * (om * om) * ce, 100.0)
            acc_posl = acc_posl + acl * posf
            acc_np = acc_np + posf
            biou_v[sl] = jnp.where(pos, 0.0, acl)
            return acc_loc, acc_posl, acc_np

        return lax.fori_loop(0, CHV, p2i, (acc_loc, acc_posl, acc_np))

    z = jnp.zeros((L,), jnp.float32)
    acc_loc, acc_posl, acc_np = lax.fori_loop(
        0, NCH, p2, (z, z, z))

    loc_sum = jnp.sum(acc_loc)
    pos_loss = jnp.sum(acc_posl)
    n_pos = jnp.sum(acc_np)
    kk = jnp.minimum(jnp.float32(A) - n_pos, n_pos * NEG_POS_RATIO)

    ones = jnp.full((L,), 1.0, jnp.float32)
    def mx(c, acc):
        return jnp.maximum(acc, biou_v[pl.ds(c * L, L)])

    rlo = jnp.float32(0.0)
    rhi = jnp.max(lax.fori_loop(0, NV, mx, z)) * 1.000001 + 1e-20
    cbase = jnp.float32(0.0)
    for _level in range(NLEVELS):
        w = (rhi - rlo) * (1.0 / NBINS)
        inv_w = (1.0 / jnp.full((L,), w, jnp.float32))[0]

        def zh(b, carry):
            hist_v[pl.ds(b * L, L)] = jnp.zeros((L,), jnp.float32)
            return carry

        lax.fori_loop(0, NBINS, zh, jnp.int32(0))

        def hp(c, carry, rlo=rlo, rhi=rhi, inv_w=inv_w):
            v = biou_v[pl.ds(c * L, L)]
            b = lax.convert_element_type((v - rlo) * inv_w, jnp.int32)
            m = jnp.logical_and(v >= rlo, b <= NBINS - 1)
            bc = jnp.minimum(jnp.maximum(b, 0), NBINS - 1)
            plsc.addupdate_scatter(hist_v, [bc * L + iot],
                                   jnp.where(m, 1.0, 0.0))
            return carry

        lax.fori_loop(0, NV, hp, jnp.int32(0))

        def cross(i, carry, cbase=cbase):
            cum, above, bstar, found = carry
            b = NBINS - 1 - i
            cb2 = jnp.sum(hist_v[pl.ds(b * L, L)])
            newcum = cum + cb2
            fire = jnp.logical_and(found < 0.5, cbase + newcum >= kk)
            above = jnp.where(fire, cum, above)
            bstar = jnp.where(fire, b, bstar)
            found = jnp.where(fire, 1.0, found)
            return newcum, above, bstar, found

        _, above, bstar, _ = lax.fori_loop(
            0, NBINS, cross,
            (jnp.float32(0.0), jnp.float32(0.0), jnp.int32(0),
             jnp.float32(0.0)))
        cbase = cbase + above
        bstarf = lax.convert_element_type(bstar, jnp.float32)
        rhi = rlo + (bstarf + 1.0) * w
        rlo = rlo + bstarf * w
    t = rlo

    def sb(c, acc):
        s, cnt = acc
        v = biou_v[pl.ds(c * L, L)]
        sel = v > t
        return s + jnp.where(sel, v, 0.0), cnt + jnp.where(sel, 1.0, 0.0)

    sv, cv = lax.fori_loop(0, NV, sb, (z, z))
    s_top = jnp.sum(sv) + t * (kk - jnp.sum(cv))
    conf_sum = pos_loss + jnp.where(kk >= 0.5, s_top, 0.0)

    outv = jnp.where(iot == 0, loc_sum,
                     jnp.where(iot == 1, conf_sum,
                               jnp.where(iot == 2, n_pos, 0.0)))
    out_v[:] = outv
    pltpu.sync_copy(out_v, out_hbm.at[pl.ds(wid * L, L)])


@jax.jit
def _run(bbox_t, conf, anch_t, gt_t):
    mesh = plsc.VectorSubcoreMesh(core_axis_name="c", subcore_axis_name="s")
    f = pl.kernel(
        _sc_body,
        mesh=mesh,
        compiler_params=pltpu.CompilerParams(needs_layout_passes=False),
        out_type=jax.ShapeDtypeStruct((B * L,), jnp.float32),
        scratch_types=[
            pltpu.VMEM((4, A), jnp.float32),
            pltpu.VMEM((A,), jnp.float32),
            pltpu.VMEM((A,), jnp.int32),
            pltpu.VMEM((G, L), jnp.float32),
            pltpu.VMEM((G, L), jnp.float32),
            pltpu.VMEM((G, L), jnp.int32),
            pltpu.VMEM((L,), jnp.float32),
            pltpu.VMEM((L,), jnp.float32),
            pltpu.VMEM((L,), jnp.float32),
            pltpu.VMEM((L,), jnp.float32),
            pltpu.VMEM((CH,), jnp.float32),
            pltpu.VMEM((L,), jnp.float32),
            pltpu.VMEM((NBINS * L,), jnp.float32),
        ],
    )
    return f(bbox_t, conf, anch_t, gt_t)


def kernel(bbox_pred, conf_pred, anchors, gt_boxes):
    bbox_t = jnp.transpose(bbox_pred, (0, 2, 1))
    anch_t = jnp.transpose(anchors, (1, 0))
    gt_t = jnp.transpose(gt_boxes, (0, 2, 1)).reshape(-1)
    conf_flat = conf_pred.reshape(-1)
    out = _run(bbox_t, conf_flat, anch_t, gt_t).reshape(B, L)
    loc = jnp.sum(out[:, 0])
    conf = jnp.sum(out[:, 1])
    n_pos = jnp.sum(out[:, 2])
    denom = jnp.maximum(n_pos, 1.0)
    return loc / denom + conf / denom


# speedup vs baseline: 1.5730x; 1.5730x over previous
import functools
import jax
import jax.numpy as jnp
from jax import lax
from jax.experimental import pallas as pl
from jax.experimental.pallas import tpu as pltpu
from jax.experimental.pallas import tpu_sc as plsc

B = 32
A = 20000
G = 16
L = 16
NV = A // L
CH = 2000
NCH = A // CH
CHV = CH // L
IOU_THRESHOLD = 0.5
NEG_POS_RATIO = 3.0
ALPHA = 0.25
LN2 = 0.6931471805599453
NBINS = 128
NLEVELS = 2


def _plog(x):
    bits = lax.bitcast_convert_type(x, jnp.int32)
    e = lax.convert_element_type((bits >> 23) - 127, jnp.float32)
    m = lax.bitcast_convert_type(
        (bits & jnp.int32(0x7FFFFF)) | jnp.int32(0x3F800000), jnp.float32)
    f = m - 1.0
    z = f / (2.0 + f)
    z2 = z * z
    at = z * (1.0 + z2 * (1.0 / 3 + z2 * (1.0 / 5 + z2 * (1.0 / 7 + z2 / 9))))
    return (e + (2.0 / LN2) * at) * LN2


def _sc_body(bbox_hbm, conf_hbm, anch_hbm, gt_hbm, out_hbm,
             slab_v, biou_v, bidx_v, gmax_v, gidx_v,
             gx1_v, gy1_v, gx2_v, gy2_v, conf_v, out_v, hist_v):
    wid = lax.axis_index("s") * 2 + lax.axis_index("c")
    iot = lax.iota(jnp.int32, L)

    pltpu.sync_copy(anch_hbm, slab_v)
    gbase = wid * (4 * G)
    pltpu.sync_copy(gt_hbm.at[pl.ds(gbase, G)], gx1_v)
    pltpu.sync_copy(gt_hbm.at[pl.ds(gbase + G, G)], gy1_v)
    pltpu.sync_copy(gt_hbm.at[pl.ds(gbase + 2 * G, G)], gx2_v)
    pltpu.sync_copy(gt_hbm.at[pl.ds(gbase + 3 * G, G)], gy2_v)

    gx1r, gy1r, gx2r, gy2r = gx1_v[:], gy1_v[:], gx2_v[:], gy2_v[:]
    gx1s = [gx1r[g] for g in range(G)]
    gy1s = [gy1r[g] for g in range(G)]
    gx2s = [gx2r[g] for g in range(G)]
    gy2s = [gy2r[g] for g in range(G)]
    ag7 = [(gx2s[g] - gx1s[g]) * (gy2s[g] - gy1s[g]) + 1e-7 for g in range(G)]

    for g in range(G):
        gmax_v[g, :] = jnp.full((L,), -1.0, jnp.float32)
        gidx_v[g, :] = jnp.zeros((L,), jnp.int32)

    def p1(c, carry):
        sl = pl.ds(c * L, L)
        ax1 = slab_v[0, sl]
        ay1 = slab_v[1, sl]
        ax2 = slab_v[2, sl]
        ay2 = slab_v[3, sl]
        area_a = (ax2 - ax1) * (ay2 - ay1)
        aid = c * L + iot
        biou = jnp.full((L,), -1.0, jnp.float32)
        bidx = jnp.zeros((L,), jnp.int32)
        for g in range(G):
            iw = jnp.maximum(
                jnp.minimum(ax2, gx2s[g]) - jnp.maximum(ax1, gx1s[g]), 0.0)
            ih = jnp.maximum(
                jnp.minimum(ay2, gy2s[g]) - jnp.maximum(ay1, gy1s[g]), 0.0)
            inter = iw * ih
            iou = inter / (area_a + ag7[g] - inter)
            upd = iou > biou
            biou = jnp.where(upd, iou, biou)
            bidx = jnp.where(upd, jnp.full((L,), g, jnp.int32), bidx)
            gm = gmax_v[g, :]
            gu = iou > gm
            gmax_v[g, :] = jnp.where(gu, iou, gm)
            gidx_v[g, :] = jnp.where(gu, aid, gidx_v[g, :])
        biou_v[sl] = biou
        bidx_v[sl] = bidx
        return carry

    lax.fori_loop(0, NV, p1, jnp.int32(0))

    fids = jnp.zeros((L,), jnp.int32)
    for g in range(G):
        row = gmax_v[g, :]
        gi = gidx_v[g, :]
        gm = jnp.max(row)
        cand = jnp.where(row == gm, gi, jnp.full((L,), A, jnp.int32))
        fid = jnp.min(cand)
        fids = jnp.where(iot == g, fid, fids)
    plsc.store_scatter(biou_v, [fids], jnp.full((L,), 2.0, jnp.float32))

    pltpu.sync_copy(bbox_hbm.at[wid], slab_v)

    def p2(k, carry):
        acc_loc, acc_posl, acc_np, maxn = carry
        pltpu.sync_copy(conf_hbm.at[pl.ds(wid * A + k * CH, CH)], conf_v)

        def p2i(j, icarry):
            acc_loc, acc_posl, acc_np, maxn = icarry
            o = k * CH + j * L
            sl = pl.ds(o, L)
            slc = pl.ds(j * L, L)
            bi = biou_v[sl]
            bx = bidx_v[sl]
            pos = bi > IOU_THRESHOLD
            posf = jnp.where(pos, 1.0, 0.0)
            tx1 = plsc.load_gather(gx1_v, [bx])
            ty1 = plsc.load_gather(gy1_v, [bx])
            tx2 = plsc.load_gather(gx2_v, [bx])
            ty2 = plsc.load_gather(gy2_v, [bx])
            px1 = slab_v[0, sl]
            py1 = slab_v[1, sl]
            px2 = slab_v[2, sl]
            py2 = slab_v[3, sl]
            area_p = (px2 - px1) * (py2 - py1)
            area_t = (tx2 - tx1) * (ty2 - ty1)
            iw = jnp.maximum(jnp.minimum(px2, tx2) - jnp.maximum(px1, tx1), 0.0)
            ih = jnp.maximum(jnp.minimum(py2, ty2) - jnp.maximum(py1, ty1), 0.0)
            inter = iw * ih
            union = area_p + area_t - inter + 1e-7
            iou = inter / union
            dx = (px1 + px2) - (tx1 + tx2)
            dy = (py1 + py2) - (ty1 + ty2)
            rho2 = 0.25 * (dx * dx + dy * dy)
            ex = jnp.maximum(px2, tx2) - jnp.minimum(px1, tx1)
            ey = jnp.maximum(py2, ty2) - jnp.minimum(py1, ty1)
            c2 = ex * ex + ey * ey + 1e-7
            loc = jnp.minimum(1.0 - iou + rho2 / c2, 100.0)
            acc_loc = acc_loc + loc * posf
            lg = conf_v[slc]
            ez = jnp.exp(-jnp.abs(lg))
            ce = jnp.maximum(lg, 0.0) - lg * posf + _plog(1.0 + ez)
            p = jnp.where(lg >= 0.0, 1.0, ez) / (1.0 + ez)
            p_t = p * posf + (1.0 - p) * (1.0 - posf)
            alpha_t = ALPHA * posf + (1.0 - ALPHA) * (1.0 - posf)
            om = 1.0 - p_t
            acl = jnp.minimum(alpha_t * (om * om) * ce, 100.0)
            acc_posl = acc_posl + acl * posf
            acc_np = acc_np + posf
            neg = jnp.where(pos, 0.0, acl)
            maxn = jnp.maximum(maxn, neg)
            biou_v[sl] = neg
            return acc_loc, acc_posl, acc_np, maxn

        return lax.fori_loop(0, CHV, p2i, (acc_loc, acc_posl, acc_np, maxn))

    z = jnp.zeros((L,), jnp.float32)
    acc_loc, acc_posl, acc_np, maxn = lax.fori_loop(
        0, NCH, p2, (z, z, z, z))

    loc_sum = jnp.sum(acc_loc)
    pos_loss = jnp.sum(acc_posl)
    n_pos = jnp.sum(acc_np)
    kk = jnp.minimum(jnp.float32(A) - n_pos, n_pos * NEG_POS_RATIO)

    ones = jnp.full((L,), 1.0, jnp.float32)
    rlo = jnp.float32(0.0)
    rhi = jnp.max(maxn) * 1.000001 + 1e-20
    cbase = jnp.float32(0.0)
    for _level in range(NLEVELS):
        w = (rhi - rlo) * (1.0 / NBINS)
        inv_w = (1.0 / jnp.full((L,), w, jnp.float32))[0]

        def zh(b, carry):
            hist_v[pl.ds(b * L, L)] = jnp.zeros((L,), jnp.float32)
            return carry

        lax.fori_loop(0, NBINS, zh, jnp.int32(0))

        def hp(c, carry, rlo=rlo, rhi=rhi, inv_w=inv_w):
            v = biou_v[pl.ds(c * L, L)]
            b = lax.convert_element_type((v - rlo) * inv_w, jnp.int32)
            m = jnp.logical_and(v >= rlo, b <= NBINS - 1)
            bc = jnp.minimum(jnp.maximum(b, 0), NBINS - 1)
            plsc.addupdate_scatter(hist_v, [bc * L + iot],
                                   jnp.where(m, 1.0, 0.0))
            return carry

        lax.fori_loop(0, NV, hp, jnp.int32(0))

        def cross(i, carry, cbase=cbase):
            cum, above, bstar, found = carry
            b = NBINS - 1 - i
            cb2 = jnp.sum(hist_v[pl.ds(b * L, L)])
            newcum = cum + cb2
            fire = jnp.logical_and(found < 0.5, cbase + newcum >= kk)
            above = jnp.where(fire, cum, above)
            bstar = jnp.where(fire, b, bstar)
            found = jnp.where(fire, 1.0, found)
            return newcum, above, bstar, found

        _, above, bstar, _ = lax.fori_loop(
            0, NBINS, cross,
            (jnp.float32(0.0), jnp.float32(0.0), jnp.int32(0),
             jnp.float32(0.0)))
        cbase = cbase + above
        bstarf = lax.convert_element_type(bstar, jnp.float32)
        rhi = rlo + (bstarf + 1.0) * w
        rlo = rlo + bstarf * w
    t = rlo

    def sb(c, acc):
        s, cnt = acc
        v = biou_v[pl.ds(c * L, L)]
        sel = v > t
        return s + jnp.where(sel, v, 0.0), cnt + jnp.where(sel, 1.0, 0.0)

    sv, cv = lax.fori_loop(0, NV, sb, (z, z))
    s_top = jnp.sum(sv) + t * (kk - jnp.sum(cv))
    conf_sum = pos_loss + jnp.where(kk >= 0.5, s_top, 0.0)

    outv = jnp.where(iot == 0, loc_sum,
                     jnp.where(iot == 1, conf_sum,
                               jnp.where(iot == 2, n_pos, 0.0)))
    out_v[:] = outv
    pltpu.sync_copy(out_v, out_hbm.at[pl.ds(wid * L, L)])


@jax.jit
def _run(bbox_t, conf, anch_t, gt_t):
    mesh = plsc.VectorSubcoreMesh(core_axis_name="c", subcore_axis_name="s")
    f = pl.kernel(
        _sc_body,
        mesh=mesh,
        compiler_params=pltpu.CompilerParams(needs_layout_passes=False),
        out_type=jax.ShapeDtypeStruct((B * L,), jnp.float32),
        scratch_types=[
            pltpu.VMEM((4, A), jnp.float32),
            pltpu.VMEM((A,), jnp.float32),
            pltpu.VMEM((A,), jnp.int32),
            pltpu.VMEM((G, L), jnp.float32),
            pltpu.VMEM((G, L), jnp.int32),
            pltpu.VMEM((L,), jnp.float32),
            pltpu.VMEM((L,), jnp.float32),
            pltpu.VMEM((L,), jnp.float32),
            pltpu.VMEM((L,), jnp.float32),
            pltpu.VMEM((CH,), jnp.float32),
            pltpu.VMEM((L,), jnp.float32),
            pltpu.VMEM((NBINS * L,), jnp.float32),
        ],
    )
    return f(bbox_t, conf, anch_t, gt_t)


def kernel(bbox_pred, conf_pred, anchors, gt_boxes):
    bbox_t = jnp.transpose(bbox_pred, (0, 2, 1))
    anch_t = jnp.transpose(anchors, (1, 0))
    gt_t = jnp.transpose(gt_boxes, (0, 2, 1)).reshape(-1)
    conf_flat = conf_pred.reshape(-1)
    out = _run(bbox_t, conf_flat, anch_t, gt_t).reshape(B, L)
    loc = jnp.sum(out[:, 0])
    conf = jnp.sum(out[:, 1])
    n_pos = jnp.sum(out[:, 2])
    denom = jnp.maximum(n_pos, 1.0)
    return loc / denom + conf / denom
